# Initial kernel scaffold; baseline (speedup 1.0000x reference)
import jax
import jax.numpy as jnp
from jax.experimental import pallas as pl


def kernel(scores, k):
    raise NotImplementedError("write your pallas kernel here")


# speedup vs baseline: 9.8175x; 9.8175x over previous
import functools

import jax
import jax.numpy as jnp
from jax.experimental import pallas as pl
from jax.experimental.pallas import tpu as pltpu

_ROWS_PER_BLOCK = 256
_NEG_INF = float("-inf")


def _topk_mask_body(k_ref, x_ref, o_ref):
    kk = k_ref[0, 0]
    x = x_ref[...]
    r, c = x.shape
    b = jax.lax.bitcast_convert_type(x, jnp.int32)
    key = b ^ ((b >> 31) & jnp.int32(0x7FFFFFFF))

    def count_ge(cand):
        return jnp.sum((key >= cand).astype(jnp.int32), axis=1, keepdims=True)

    t = jnp.where(count_ge(jnp.zeros((r, 1), jnp.int32)) >= kk,
                  jnp.int32(0), jnp.int32(-2147483648))
    for bit in range(30, -1, -1):
        cand = t | jnp.int32(1 << bit)
        t = jnp.where(count_ge(cand) >= kk, cand, t)

    gt = key > t
    eq = key == t
    cnt_gt = jnp.sum(gt.astype(jnp.int32), axis=1, keepdims=True)
    need = kk - cnt_gt

    idx = jax.lax.broadcasted_iota(jnp.int32, (r, c), 1)
    eq_i = eq.astype(jnp.int32)
    i_cut = jnp.zeros((r, 1), jnp.int32)
    for bit in range(9, -1, -1):
        cand = i_cut + jnp.int32((1 << bit) - 1)
        cnt = jnp.sum(jnp.where(idx <= cand, eq_i, 0), axis=1, keepdims=True)
        i_cut = jnp.where(cnt >= need, i_cut, i_cut + jnp.int32(1 << bit))

    keep = gt | (eq & (idx <= i_cut))
    o_ref[...] = jnp.where(keep, x, _NEG_INF)


@jax.jit
def kernel(scores, k):
    b, s, c = scores.shape
    n = b * s
    x = scores.reshape(n, c)
    kk = jnp.clip(k, 1, c).astype(jnp.int32).reshape(1, 1)
    out = pl.pallas_call(
        _topk_mask_body,
        grid=(n // _ROWS_PER_BLOCK,),
        in_specs=[
            pl.BlockSpec(memory_space=pltpu.SMEM),
            pl.BlockSpec((_ROWS_PER_BLOCK, c), lambda i: (i, 0)),
        ],
        out_specs=pl.BlockSpec((_ROWS_PER_BLOCK, c), lambda i: (i, 0)),
        out_shape=jax.ShapeDtypeStruct((n, c), scores.dtype),
        compiler_params=pltpu.CompilerParams(
            dimension_semantics=("arbitrary",),
        ),
    )(kk, x)
    return out.reshape(b, s, c)


# speedup vs baseline: 16.4275x; 1.6733x over previous
import functools

import jax
import jax.numpy as jnp
from jax.experimental import pallas as pl
from jax.experimental.pallas import tpu as pltpu

_ROWS_PER_BLOCK = 256
_NEG_INF = float("-inf")


def _topk_mask_body(k_ref, x_ref, o_ref):
    kk = k_ref[0, 0]
    kf = kk.astype(jnp.float32)
    x = x_ref[...]
    r, c = x.shape
    b = jax.lax.bitcast_convert_type(x, jnp.int32)
    key = b ^ ((b >> 31) & jnp.int32(0x7FFFFFFF))

    def count_ge(cand):
        return jnp.sum(jnp.where(key >= cand, 1.0, 0.0), axis=1, keepdims=True)

    t = jnp.where(count_ge(jnp.zeros((r, 1), jnp.int32)) >= kf,
                  jnp.int32(0), jnp.int32(-2147483648))
    for bit in range(30, -1, -1):
        cand = t | jnp.int32(1 << bit)
        t = jnp.where(count_ge(cand) >= kf, cand, t)

    o_ref[...] = jnp.where(key >= t, x, _NEG_INF)

    any_tie = jnp.any(count_ge(t) > kf)

    @pl.when(any_tie)
    def _tie_fixup():
        gt = key > t
        eq = key == t
        cnt_gt = jnp.sum(jnp.where(gt, 1.0, 0.0), axis=1, keepdims=True)
        need = kf - cnt_gt
        idx = jax.lax.broadcasted_iota(jnp.int32, (r, c), 1)
        eq_f = jnp.where(eq, 1.0, 0.0)
        i_cut = jnp.zeros((r, 1), jnp.int32)
        for bit in range(9, -1, -1):
            cand = i_cut + jnp.int32((1 << bit) - 1)
            cnt = jnp.sum(jnp.where(idx <= cand, eq_f, 0.0),
                          axis=1, keepdims=True)
            i_cut = jnp.where(cnt >= need, i_cut, i_cut + jnp.int32(1 << bit))
        keep = gt | (eq & (idx <= i_cut))
        o_ref[...] = jnp.where(keep, x, _NEG_INF)


@jax.jit
def kernel(scores, k):
    b, s, c = scores.shape
    n = b * s
    x = scores.reshape(n, c)
    kk = jnp.clip(k, 1, c).astype(jnp.int32).reshape(1, 1)
    out = pl.pallas_call(
        _topk_mask_body,
        grid=(n // _ROWS_PER_BLOCK,),
        in_specs=[
            pl.BlockSpec(memory_space=pltpu.SMEM),
            pl.BlockSpec((_ROWS_PER_BLOCK, c), lambda i: (i, 0)),
        ],
        out_specs=pl.BlockSpec((_ROWS_PER_BLOCK, c), lambda i: (i, 0)),
        out_shape=jax.ShapeDtypeStruct((n, c), scores.dtype),
        compiler_params=pltpu.CompilerParams(
            dimension_semantics=("arbitrary",),
        ),
    )(kk, x)
    return out.reshape(b, s, c)
